# baseline (speedup 1.0000x reference)
import functools

import jax
import jax.numpy as jnp
import numpy as np
from jax import lax
from jax.experimental import pallas as pl
from jax.experimental.pallas import tpu as pltpu
from jax.experimental.pallas import tpu_sc as plsc

N = 10000
E = 320000
D = 128

NC = 2
NS = 16
NW = NC * NS
CHUNK = 40
NCHG = E // CHUNK
FULL_T = NCHG // NW
MACRO = 10
NMACRO = FULL_T // MACRO
MPAIRS = NMACRO // 2
OUT_CHUNK = 40
N_OUT_CHUNKS = N // OUT_CHUNK
OUT_ROUNDS = -(-N_OUT_CHUNKS // NS)

ROW_BLK = 1000
GRID = N // ROW_BLK


def _proj_body(x_ref, w_ref, b_ref, k_ref, q_ref, v_ref, s_ref):
    acc = jnp.dot(x_ref[...], w_ref[...], preferred_element_type=jnp.float32)
    acc = acc + b_ref[...]
    k_ref[...] = acc[:, 0 * D:1 * D]
    q_ref[...] = acc[:, 1 * D:2 * D]
    v_ref[...] = acc[:, 2 * D:3 * D]
    s_ref[...] = acc[:, 3 * D:4 * D]


def _edge_body(k_hbm, qv_hbm, eim_hbm, agg_hbm,
               kda, qsa, kdb, qsb, msga, msgb,
               ima, imb, agg_sh, sema, semb, semia, semib, semsa, semsb):
    c = lax.axis_index("c")
    s = lax.axis_index("s")
    wid = c * NS + s
    stage = kda.at[pl.ds(0, OUT_CHUNK)]

    def _zrow(i, _):
        r = i // 8
        j = (i % 8) * 16
        stage[r, pl.ds(j, 16)] = jnp.zeros((16,), jnp.float32)
        return 0
    lax.fori_loop(0, OUT_CHUNK * 8, _zrow, 0)
    for g in range(OUT_ROUNDS):
        blk = g * NS + s

        @pl.when(blk < N_OUT_CHUNKS)
        def _zero_blk():
            pltpu.sync_copy(stage, agg_sh.at[pl.ds(blk * OUT_CHUNK,
                                                   OUT_CHUNK)])
    plsc.subcore_barrier()

    bufa = (kda, qsa, msga, sema, semsa)
    bufb = (kdb, qsb, msgb, semb, semsb)

    def _fire(im, j, buf):
        kd, qv, msg, sem, sems = buf
        pltpu.async_copy(k_hbm.at[im.at[j, 1]], kd, sem)
        pltpu.async_copy(qv_hbm.at[im.at[j, 0]], qv, sem)

    def _finish(im, j, buf, scatter_guard=None):
        kd, qv, msg, sem, sems = buf
        pltpu.make_async_copy(k_hbm.at[im.at[j, 1]], kd, sem).wait()
        pltpu.make_async_copy(qv_hbm.at[im.at[j, 0]], qv, sem).wait()

        def _drain():
            pltpu.make_async_copy(msg, agg_sh.at[im.at[j, 1]], sems).wait()
        if scatter_guard is None:
            _drain()
        else:
            pl.when(scatter_guard)(_drain)

        hi_mask = jnp.int32(-65536)

        def _unpk(w):
            lo = lax.bitcast_convert_type(w << 16, jnp.float32)
            hi = lax.bitcast_convert_type(w & hi_mask, jnp.float32)
            return lo, hi

        def _row(r, _):
            for g in range(4):
                k_lo = kd[r, pl.ds(32 * g, 16)]
                k_hi = kd[r, pl.ds(32 * g + 16, 16)]
                q_lo, q_hi = _unpk(qv[r, pl.ds(16 * g, 16)])
                v_lo, v_hi = _unpk(qv[r, pl.ds(64 + 16 * g, 16)])
                for kk, qq, vv, off in ((k_lo, q_lo, v_lo, 32 * g),
                                        (k_hi, q_hi, v_hi, 32 * g + 16)):
                    z = kk + qq
                    eta = 1.0 / (1.0 + jnp.exp(-z))
                    msg[r, pl.ds(off, 16)] = eta * vv
            return 0
        lax.fori_loop(0, CHUNK, _row, 0)

        pltpu.async_copy(msg, agg_sh.at[im.at[j, 1]], sems, add=True)

    def _buf(j):
        return bufa if j % 2 == 0 else bufb

    def _macro_section(im, fire_next_first, hooks, guards):
        for j in range(MACRO):
            if j < MACRO - 1:
                _fire(im, j + 1, _buf(j + 1))
            else:
                fire_next_first()
            _finish(im, j, _buf(j), guards.get(j))
            if j in hooks:
                hooks[j]()

    pltpu.sync_copy(eim_hbm.at[0, wid], ima)
    _fire(ima, 0, bufa)

    def _mpair(u, _):
        def _pf_b():
            pltpu.async_copy(eim_hbm.at[2 * u + 1, wid], imb, semib)

        def _start_m1():
            pltpu.make_async_copy(eim_hbm.at[0, wid], imb, semib).wait()
            _fire(imb, 0, bufa)
        nz = u > 0
        _macro_section(ima, _start_m1, {1: _pf_b}, {0: nz, 1: nz})

        def _pf_a():
            pltpu.async_copy(eim_hbm.at[2 * u + 2, wid], ima, semia)

        def _start_m2():
            pltpu.make_async_copy(eim_hbm.at[0, wid], ima, semia).wait()
            _fire(ima, 0, bufa)
        _macro_section(imb, _start_m2, {1: _pf_a}, {})
        return 0
    lax.fori_loop(0, MPAIRS, _mpair, 0)

    _macro_section(ima, lambda: None, {}, {})
    pltpu.make_async_copy(msga, agg_sh.at[ima.at[0, 1]], semsa).wait()
    pltpu.make_async_copy(msgb, agg_sh.at[ima.at[1, 1]], semsb).wait()

    plsc.subcore_barrier()
    for g in range(OUT_ROUNDS):
        blk = g * NS + s

        @pl.when(blk < N_OUT_CHUNKS)
        def _copy_blk():
            off = blk * OUT_CHUNK
            pltpu.sync_copy(agg_sh.at[pl.ds(off, OUT_CHUNK)], stage)
            pltpu.sync_copy(stage, agg_hbm.at[c, pl.ds(off, OUT_CHUNK)])


def _stats_body(agg_ref, skip_ref, h_ref, sum_ref, sq_ref):
    h = agg_ref[0] + agg_ref[1] + skip_ref[...]
    h_ref[...] = h

    @pl.when(pl.program_id(0) == 0)
    def _init():
        sum_ref[...] = jnp.zeros_like(sum_ref)
        sq_ref[...] = jnp.zeros_like(sq_ref)

    sum_ref[...] += jnp.sum(h, axis=0, keepdims=True)
    sq_ref[...] += jnp.sum(h * h, axis=0, keepdims=True)


def _norm_body(h_ref, sum_ref, sq_ref, gamma_ref, beta_ref, x_ref, o_ref):
    inv_n = 1.0 / N
    mean = sum_ref[...] * inv_n
    var = sq_ref[...] * inv_n - mean * mean
    scale = gamma_ref[...] * lax.rsqrt(var + 1e-5)
    h = (h_ref[...] - mean) * scale + beta_ref[...]
    o_ref[...] = jnp.maximum(h, 0.0) + x_ref[...]


_PERM = np.empty((D,), dtype=np.int32)
for _g in range(D // 32):
    for _i in range(16):
        _PERM[32 * _g + 2 * _i] = 32 * _g + _i
        _PERM[32 * _g + 2 * _i + 1] = 32 * _g + 16 + _i


def kernel(x, edge_index, Wk, bk, Wq, bq, Wv, bv, Wskip, bias, gamma, beta):
    w_all = jnp.concatenate(
        [Wk.T, Wq[_PERM].T, Wv[_PERM].T, Wskip.T], axis=1)
    b_all = jnp.concatenate(
        [bk, bq[_PERM], bv[_PERM], bias], axis=0).reshape(1, 4 * D)

    tab = jax.ShapeDtypeStruct((N, D), jnp.float32)
    k, q, v, skip = pl.pallas_call(
        _proj_body,
        grid=(GRID,),
        in_specs=[
            pl.BlockSpec((ROW_BLK, D), lambda i: (i, 0)),
            pl.BlockSpec((D, 4 * D), lambda i: (0, 0)),
            pl.BlockSpec((1, 4 * D), lambda i: (0, 0)),
        ],
        out_specs=[pl.BlockSpec((ROW_BLK, D), lambda i: (i, 0))] * 4,
        out_shape=[tab, tab, tab, tab],
    )(x, w_all, b_all)

    def _pack32(t):
        tb = t.astype(jnp.bfloat16).reshape(N, D // 2, 2)
        return jax.lax.bitcast_convert_type(tb, jnp.int32)

    qv32 = jnp.concatenate([_pack32(q), _pack32(v)], axis=1)

    src = edge_index[0]
    dst = edge_index[1]
    ei = jnp.stack([src.reshape(FULL_T, NW, CHUNK),
                    dst.reshape(FULL_T, NW, CHUNK)], axis=2)
    eim = ei.reshape(NMACRO, MACRO, NW, 2, CHUNK).transpose(0, 2, 1, 3, 4)

    mesh = plsc.VectorSubcoreMesh(
        core_axis_name="c", subcore_axis_name="s",
        num_cores=NC, num_subcores=NS)
    edge_fn = pl.kernel(
        _edge_body,
        out_type=jax.ShapeDtypeStruct((NC, N, D), jnp.float32),
        mesh=mesh,
        scratch_types=[
            pltpu.VMEM((CHUNK, D), jnp.float32),
            pltpu.VMEM((CHUNK, D), jnp.int32),
            pltpu.VMEM((CHUNK, D), jnp.float32),
            pltpu.VMEM((CHUNK, D), jnp.int32),
            pltpu.VMEM((CHUNK, D), jnp.float32),
            pltpu.VMEM((CHUNK, D), jnp.float32),
            pltpu.VMEM((MACRO, 2, CHUNK), jnp.int32),
            pltpu.VMEM((MACRO, 2, CHUNK), jnp.int32),
            pltpu.VMEM_SHARED((N, D), jnp.float32),
            pltpu.SemaphoreType.DMA,
            pltpu.SemaphoreType.DMA,
            pltpu.SemaphoreType.DMA,
            pltpu.SemaphoreType.DMA,
            pltpu.SemaphoreType.DMA,
            pltpu.SemaphoreType.DMA,
        ],
    )
    agg = edge_fn(k, qv32, eim)

    h, hsum, hsq = pl.pallas_call(
        _stats_body,
        grid=(GRID,),
        in_specs=[
            pl.BlockSpec((NC, ROW_BLK, D), lambda i: (0, i, 0)),
            pl.BlockSpec((ROW_BLK, D), lambda i: (i, 0)),
        ],
        out_specs=[
            pl.BlockSpec((ROW_BLK, D), lambda i: (i, 0)),
            pl.BlockSpec((1, D), lambda i: (0, 0)),
            pl.BlockSpec((1, D), lambda i: (0, 0)),
        ],
        out_shape=[
            jax.ShapeDtypeStruct((N, D), jnp.float32),
            jax.ShapeDtypeStruct((1, D), jnp.float32),
            jax.ShapeDtypeStruct((1, D), jnp.float32),
        ],
    )(agg, skip)

    out = pl.pallas_call(
        _norm_body,
        grid=(GRID,),
        in_specs=[
            pl.BlockSpec((ROW_BLK, D), lambda i: (i, 0)),
            pl.BlockSpec((1, D), lambda i: (0, 0)),
            pl.BlockSpec((1, D), lambda i: (0, 0)),
            pl.BlockSpec((1, D), lambda i: (0, 0)),
            pl.BlockSpec((1, D), lambda i: (0, 0)),
            pl.BlockSpec((ROW_BLK, D), lambda i: (i, 0)),
        ],
        out_specs=pl.BlockSpec((ROW_BLK, D), lambda i: (i, 0)),
        out_shape=jax.ShapeDtypeStruct((N, D), jnp.float32),
    )(h, hsum, hsq, gamma.reshape(1, D), beta.reshape(1, D), x)

    return out


# speedup vs baseline: 1.1087x; 1.1087x over previous
import functools

import jax
import jax.numpy as jnp
import numpy as np
from jax import lax
from jax.experimental import pallas as pl
from jax.experimental.pallas import tpu as pltpu
from jax.experimental.pallas import tpu_sc as plsc

N = 10000
E = 320000
D = 128

NC = 2
NS = 16
NW = NC * NS
CHUNK = 40
NCHG = E // CHUNK
FULL_T = NCHG // NW
MACRO = 10
NMACRO = FULL_T // MACRO
MPAIRS = NMACRO // 2
OUT_CHUNK = 40
N_OUT_CHUNKS = N // OUT_CHUNK
OUT_ROUNDS = -(-N_OUT_CHUNKS // NS)

ROW_BLK = 1000
GRID = N // ROW_BLK


def _proj_body(x_ref, w_ref, b_ref, k_ref, q_ref, v_ref, s_ref):
    acc = jnp.dot(x_ref[...], w_ref[...], preferred_element_type=jnp.float32)
    acc = acc + b_ref[...]
    k_ref[...] = acc[:, 0 * D:1 * D]
    q_ref[...] = acc[:, 1 * D:2 * D]
    v_ref[...] = acc[:, 2 * D:3 * D]
    s_ref[...] = acc[:, 3 * D:4 * D]


def _edge_body(k_hbm, q_hbm, v_hbm, eim_hbm, agg_hbm,
               kda, qsa, vsa, kdb, qsb, vsb, msga, msgb,
               ima, imb, agg_sh, sema, semb, semia, semib, semsa, semsb):
    c = lax.axis_index("c")
    s = lax.axis_index("s")
    wid = c * NS + s
    stage = kda.at[pl.ds(0, OUT_CHUNK)]

    def _zrow(i, _):
        r = i // 8
        j = (i % 8) * 16
        stage[r, pl.ds(j, 16)] = jnp.zeros((16,), jnp.float32)
        return 0
    lax.fori_loop(0, OUT_CHUNK * 8, _zrow, 0)
    for g in range(OUT_ROUNDS):
        blk = g * NS + s

        @pl.when(blk < N_OUT_CHUNKS)
        def _zero_blk():
            pltpu.sync_copy(stage, agg_sh.at[pl.ds(blk * OUT_CHUNK,
                                                   OUT_CHUNK)])
    plsc.subcore_barrier()

    bufa = (kda, qsa, vsa, msga, sema, semsa)
    bufb = (kdb, qsb, vsb, msgb, semb, semsb)

    def _fire(im, j, buf):
        kd, qs, vs, msg, sem, sems = buf
        pltpu.async_copy(k_hbm.at[im.at[j, 1]], kd, sem)
        pltpu.async_copy(q_hbm.at[im.at[j, 0]], qs, sem)
        pltpu.async_copy(v_hbm.at[im.at[j, 0]], vs, sem)

    def _finish(im, j, buf, scatter_guard=None):
        kd, qs, vs, msg, sem, sems = buf
        pltpu.make_async_copy(k_hbm.at[im.at[j, 1]], kd, sem).wait()
        pltpu.make_async_copy(q_hbm.at[im.at[j, 0]], qs, sem).wait()
        pltpu.make_async_copy(v_hbm.at[im.at[j, 0]], vs, sem).wait()

        def _drain():
            pltpu.make_async_copy(msg, agg_sh.at[im.at[j, 1]], sems).wait()
        if scatter_guard is None:
            _drain()
        else:
            pl.when(scatter_guard)(_drain)

        def _row(r, _):
            for jj in range(8):
                sl = pl.ds(jj * 16, 16)
                t = jnp.exp(kd[r, sl] + qs[r, sl])
                msg[r, sl] = vs[r, sl] / (1.0 + t)
            return 0
        lax.fori_loop(0, CHUNK, _row, 0)

        pltpu.async_copy(msg, agg_sh.at[im.at[j, 1]], sems, add=True)

    def _buf(j):
        return bufa if j % 2 == 0 else bufb

    def _macro_section(im, fire_next_first, hooks, guards):
        for j in range(MACRO):
            if j < MACRO - 1:
                _fire(im, j + 1, _buf(j + 1))
            else:
                fire_next_first()
            _finish(im, j, _buf(j), guards.get(j))
            if j in hooks:
                hooks[j]()

    pltpu.sync_copy(eim_hbm.at[0, wid], ima)
    _fire(ima, 0, bufa)

    def _mpair(u, _):
        def _pf_b():
            pltpu.async_copy(eim_hbm.at[2 * u + 1, wid], imb, semib)

        def _start_m1():
            pltpu.make_async_copy(eim_hbm.at[0, wid], imb, semib).wait()
            _fire(imb, 0, bufa)
        nz = u > 0
        _macro_section(ima, _start_m1, {1: _pf_b}, {0: nz, 1: nz})

        def _pf_a():
            pltpu.async_copy(eim_hbm.at[2 * u + 2, wid], ima, semia)

        def _start_m2():
            pltpu.make_async_copy(eim_hbm.at[0, wid], ima, semia).wait()
            _fire(ima, 0, bufa)
        _macro_section(imb, _start_m2, {1: _pf_a}, {})
        return 0
    lax.fori_loop(0, MPAIRS, _mpair, 0)

    _macro_section(ima, lambda: None, {}, {})
    pltpu.make_async_copy(msga, agg_sh.at[ima.at[0, 1]], semsa).wait()
    pltpu.make_async_copy(msgb, agg_sh.at[ima.at[1, 1]], semsb).wait()

    plsc.subcore_barrier()
    for g in range(OUT_ROUNDS):
        blk = g * NS + s

        @pl.when(blk < N_OUT_CHUNKS)
        def _copy_blk():
            off = blk * OUT_CHUNK
            pltpu.sync_copy(agg_sh.at[pl.ds(off, OUT_CHUNK)], stage)
            pltpu.sync_copy(stage, agg_hbm.at[c, pl.ds(off, OUT_CHUNK)])


def _stats_body(agg_ref, skip_ref, h_ref, sum_ref, sq_ref):
    h = agg_ref[0] + agg_ref[1] + skip_ref[...]
    h_ref[...] = h

    @pl.when(pl.program_id(0) == 0)
    def _init():
        sum_ref[...] = jnp.zeros_like(sum_ref)
        sq_ref[...] = jnp.zeros_like(sq_ref)

    sum_ref[...] += jnp.sum(h, axis=0, keepdims=True)
    sq_ref[...] += jnp.sum(h * h, axis=0, keepdims=True)


def _norm_body(h_ref, sum_ref, sq_ref, gamma_ref, beta_ref, x_ref, o_ref):
    inv_n = 1.0 / N
    mean = sum_ref[...] * inv_n
    var = sq_ref[...] * inv_n - mean * mean
    scale = gamma_ref[...] * lax.rsqrt(var + 1e-5)
    h = (h_ref[...] - mean) * scale + beta_ref[...]
    o_ref[...] = jnp.maximum(h, 0.0) + x_ref[...]


def kernel(x, edge_index, Wk, bk, Wq, bq, Wv, bv, Wskip, bias, gamma, beta):
    cneg = jnp.float32(-1.0)
    w_all = jnp.concatenate(
        [cneg * Wk.T, cneg * Wq.T, Wv.T, Wskip.T], axis=1)
    b_all = jnp.concatenate(
        [cneg * bk, cneg * bq, bv, bias], axis=0).reshape(1, 4 * D)

    tab = jax.ShapeDtypeStruct((N, D), jnp.float32)
    k, q, v, skip = pl.pallas_call(
        _proj_body,
        grid=(GRID,),
        in_specs=[
            pl.BlockSpec((ROW_BLK, D), lambda i: (i, 0)),
            pl.BlockSpec((D, 4 * D), lambda i: (0, 0)),
            pl.BlockSpec((1, 4 * D), lambda i: (0, 0)),
        ],
        out_specs=[pl.BlockSpec((ROW_BLK, D), lambda i: (i, 0))] * 4,
        out_shape=[tab, tab, tab, tab],
    )(x, w_all, b_all)

    src = edge_index[0]
    dst = edge_index[1]
    ei = jnp.stack([src.reshape(FULL_T, NW, CHUNK),
                    dst.reshape(FULL_T, NW, CHUNK)], axis=2)
    eim = ei.reshape(NMACRO, MACRO, NW, 2, CHUNK).transpose(0, 2, 1, 3, 4)

    mesh = plsc.VectorSubcoreMesh(
        core_axis_name="c", subcore_axis_name="s",
        num_cores=NC, num_subcores=NS)
    edge_fn = pl.kernel(
        _edge_body,
        out_type=jax.ShapeDtypeStruct((NC, N, D), jnp.float32),
        mesh=mesh,
        scratch_types=[
            pltpu.VMEM((CHUNK, D), jnp.float32),
            pltpu.VMEM((CHUNK, D), jnp.float32),
            pltpu.VMEM((CHUNK, D), jnp.float32),
            pltpu.VMEM((CHUNK, D), jnp.float32),
            pltpu.VMEM((CHUNK, D), jnp.float32),
            pltpu.VMEM((CHUNK, D), jnp.float32),
            pltpu.VMEM((CHUNK, D), jnp.float32),
            pltpu.VMEM((CHUNK, D), jnp.float32),
            pltpu.VMEM((MACRO, 2, CHUNK), jnp.int32),
            pltpu.VMEM((MACRO, 2, CHUNK), jnp.int32),
            pltpu.VMEM_SHARED((N, D), jnp.float32),
            pltpu.SemaphoreType.DMA,
            pltpu.SemaphoreType.DMA,
            pltpu.SemaphoreType.DMA,
            pltpu.SemaphoreType.DMA,
            pltpu.SemaphoreType.DMA,
            pltpu.SemaphoreType.DMA,
        ],
    )
    agg = edge_fn(k, q, v, eim)

    h, hsum, hsq = pl.pallas_call(
        _stats_body,
        grid=(GRID,),
        in_specs=[
            pl.BlockSpec((NC, ROW_BLK, D), lambda i: (0, i, 0)),
            pl.BlockSpec((ROW_BLK, D), lambda i: (i, 0)),
        ],
        out_specs=[
            pl.BlockSpec((ROW_BLK, D), lambda i: (i, 0)),
            pl.BlockSpec((1, D), lambda i: (0, 0)),
            pl.BlockSpec((1, D), lambda i: (0, 0)),
        ],
        out_shape=[
            jax.ShapeDtypeStruct((N, D), jnp.float32),
            jax.ShapeDtypeStruct((1, D), jnp.float32),
            jax.ShapeDtypeStruct((1, D), jnp.float32),
        ],
    )(agg, skip)

    out = pl.pallas_call(
        _norm_body,
        grid=(GRID,),
        in_specs=[
            pl.BlockSpec((ROW_BLK, D), lambda i: (i, 0)),
            pl.BlockSpec((1, D), lambda i: (0, 0)),
            pl.BlockSpec((1, D), lambda i: (0, 0)),
            pl.BlockSpec((1, D), lambda i: (0, 0)),
            pl.BlockSpec((1, D), lambda i: (0, 0)),
            pl.BlockSpec((ROW_BLK, D), lambda i: (i, 0)),
        ],
        out_specs=pl.BlockSpec((ROW_BLK, D), lambda i: (i, 0)),
        out_shape=jax.ShapeDtypeStruct((N, D), jnp.float32),
    )(h, hsum, hsq, gamma.reshape(1, D), beta.reshape(1, D), x)

    return out
